# baseline (speedup 1.0000x reference)
import jax
import jax.numpy as jnp
from jax import lax
from jax.experimental import pallas as pl
from jax.experimental.pallas import tpu as pltpu
from jax.experimental.pallas import tpu_sc as plsc

N_NODES = 10000
D_FEAT = 128
N_EDGES = 320000
NC = 2
NS = 16
NW = NC * NS
EPW = N_EDGES // NW
K = 112
R = 3
SL = 2 * R
NFULL = EPW // K
T = NFULL // SL
NEPI = NFULL - T * SL
REM = EPW - NFULL * K
ROWS_PER_TILE = (N_NODES // NS) // 8 * 8
ROWS_TAIL = N_NODES - NS * ROWS_PER_TILE


def _sc_partial_body(x_hbm, src_hbm, dst_hbm, p_hbm, *refs):
    srcs = refs[0:SL]
    dsts = refs[SL:2 * SL]
    rows = refs[2 * SL:2 * SL + R]
    srcr, dstr = refs[2 * SL + R:2 * SL + R + 2]
    acc = refs[2 * SL + R + 2]
    sems = refs[2 * SL + R + 3:]
    isems = sems[0:SL]
    gsems = sems[SL:SL + R]
    ssems = sems[SL + R:SL + 2 * R]

    cid = lax.axis_index("c")
    sid = lax.axis_index("s")
    wid = cid * NS + sid
    base = wid * EPW

    def idx_fire(s, off):
        pltpu.async_copy(src_hbm.at[pl.ds(off, K)], srcs[s], isems[s])
        pltpu.async_copy(dst_hbm.at[pl.ds(off, K)], dsts[s], isems[s])

    def idx_wait(s, off):
        pltpu.make_async_copy(src_hbm.at[pl.ds(off, K)], srcs[s], isems[s]).wait()
        pltpu.make_async_copy(dst_hbm.at[pl.ds(off, K)], dsts[s], isems[s]).wait()

    def gather_fire(k, s):
        pltpu.async_copy(x_hbm.at[srcs[s]], rows[k], gsems[k])

    def gather_wait(k, s):
        pltpu.make_async_copy(x_hbm.at[srcs[s]], rows[k], gsems[k]).wait()

    def scatter_fire(k, s):
        pltpu.async_copy(rows[k], acc.at[dsts[s]], ssems[k], add=True)

    def scatter_wait(k, s):
        pltpu.make_async_copy(rows[k], acc.at[dsts[s]], ssems[k]).wait()

    r0 = sid * ROWS_PER_TILE
    pltpu.sync_copy(x_hbm.at[pl.ds(r0, ROWS_PER_TILE)],
                    acc.at[pl.ds(r0, ROWS_PER_TILE)])

    @pl.when(sid == NS - 1)
    def _init_tail():
        t0 = NS * ROWS_PER_TILE
        pltpu.sync_copy(x_hbm.at[pl.ds(t0, ROWS_TAIL)],
                        acc.at[pl.ds(t0, ROWS_TAIL)])

    plsc.subcore_barrier()

    for s in range(SL):
        idx_fire(s, base + s * K)
    for k in range(R):
        idx_wait(k, base + k * K)
        gather_fire(k, k)

    def body(t, carry):
        c0 = base + (t * SL) * K

        for m in range(R):
            gather_wait(m, m)
            scatter_fire(m, m)
        for m in range(R):
            scatter_wait(m, m)
            idx_wait(m + R, c0 + (m + R) * K)
            gather_fire(m, m + R)

            @pl.when(t < T - 1)
            def _restage_lo():
                idx_fire(m, c0 + (m + SL) * K)

        for m in range(R):
            gather_wait(m, m + R)
            scatter_fire(m, m + R)
        for m in range(R):
            scatter_wait(m, m + R)

            @pl.when(t < T - 1)
            def _next_hi():
                idx_wait(m, c0 + (m + SL) * K)
                gather_fire(m, m)
                idx_fire(m + R, c0 + (m + R + SL) * K)

        return carry

    lax.fori_loop(0, T, body, 0)

    e0 = base + T * SL * K
    for s in range(NEPI):
        idx_fire(s, e0 + s * K)
    for s in range(R):
        idx_wait(s, e0 + s * K)
        gather_fire(s, s)
    for s in range(R, NEPI):
        idx_wait(s, e0 + s * K)
    for s in range(R):
        gather_wait(s, s)
        scatter_fire(s, s)
    for i, s in enumerate(range(R, NEPI)):
        scatter_wait(i, i)
        gather_fire(i, s)
    for i, s in enumerate(range(R, NEPI)):
        gather_wait(i, s)
        scatter_fire(i, s)
    for k in range(NEPI - R, R):
        scatter_wait(k, k)
    for i, s in enumerate(range(R, NEPI)):
        scatter_wait(i, s)

    if REM:
        off = base + NFULL * K
        pltpu.sync_copy(src_hbm.at[pl.ds(off, REM)], srcr)
        pltpu.sync_copy(dst_hbm.at[pl.ds(off, REM)], dstr)
        pltpu.async_copy(x_hbm.at[srcr], rows[0].at[pl.ds(0, REM)],
                         gsems[0]).wait()
        pltpu.sync_copy(rows[0].at[pl.ds(0, REM)], acc.at[dstr], add=True)

    plsc.subcore_barrier()
    pltpu.sync_copy(acc.at[pl.ds(r0, ROWS_PER_TILE)],
                    p_hbm.at[cid, pl.ds(r0, ROWS_PER_TILE)])

    @pl.when(sid == NS - 1)
    def _drain_tail():
        t0 = NS * ROWS_PER_TILE
        pltpu.sync_copy(acc.at[pl.ds(t0, ROWS_TAIL)],
                        p_hbm.at[cid, pl.ds(t0, ROWS_TAIL)])


def _combine_body(x_ref, p_ref, o_ref):
    o_ref[...] = p_ref[0] + p_ref[1] - x_ref[...]


def kernel(x, edge_index):
    src = edge_index[0].astype(jnp.int32)
    dst = edge_index[1].astype(jnp.int32)

    mesh = plsc.VectorSubcoreMesh(core_axis_name="c", subcore_axis_name="s",
                                  num_cores=NC, num_subcores=NS)
    p = pl.kernel(
        _sc_partial_body,
        out_type=jax.ShapeDtypeStruct((NC, N_NODES, D_FEAT), jnp.float32),
        mesh=mesh,
        scratch_types=(
            [pltpu.VMEM((K,), jnp.int32) for _ in range(2 * SL)]
            + [pltpu.VMEM((K, D_FEAT), jnp.float32) for _ in range(R)]
            + [pltpu.VMEM((REM,), jnp.int32) for _ in range(2)]
            + [pltpu.VMEM_SHARED((N_NODES, D_FEAT), jnp.float32)]
            + [pltpu.SemaphoreType.DMA for _ in range(SL + 2 * R)]
        ),
    )(x, src, dst)

    BLK = 400
    out = pl.pallas_call(
        _combine_body,
        out_shape=jax.ShapeDtypeStruct((N_NODES, D_FEAT), jnp.float32),
        grid=(N_NODES // BLK,),
        in_specs=[pl.BlockSpec((BLK, D_FEAT), lambda i: (i, 0)),
                  pl.BlockSpec((NC, BLK, D_FEAT), lambda i: (0, i, 0))],
        out_specs=pl.BlockSpec((BLK, D_FEAT), lambda i: (i, 0)),
    )(x, p)
    return out


# speedup vs baseline: 1.0641x; 1.0641x over previous
import jax
import jax.numpy as jnp
from jax import lax
from jax.experimental import pallas as pl
from jax.experimental.pallas import tpu as pltpu
from jax.experimental.pallas import tpu_sc as plsc

N_NODES = 10000
D_FEAT = 128
N_EDGES = 320000
NC = 2
NS = 16
NW = NC * NS
EPW = N_EDGES // NW
K = 80
R = 4
SL = 2 * R
NFULL = EPW // K
T = NFULL // SL
NEPI = NFULL - T * SL
ROWS_PER_TILE = (N_NODES // NS) // 8 * 8
ROWS_TAIL = N_NODES - NS * ROWS_PER_TILE


def _sc_partial_body(x_hbm, src_hbm, dst_hbm, p_hbm, *refs):
    srcs = refs[0:SL]
    dsts = refs[SL:2 * SL]
    rows = refs[2 * SL:2 * SL + R]
    acc = refs[2 * SL + R]
    sems = refs[2 * SL + R + 1:]
    isems = sems[0:SL]
    gsems = sems[SL:SL + R]
    ssems = sems[SL + R:SL + 2 * R]

    cid = lax.axis_index("c")
    sid = lax.axis_index("s")
    wid = cid * NS + sid
    base = wid * EPW

    def idx_fire(s, off):
        pltpu.async_copy(src_hbm.at[pl.ds(off, K)], srcs[s], isems[s])
        pltpu.async_copy(dst_hbm.at[pl.ds(off, K)], dsts[s], isems[s])

    def idx_wait(s, off):
        pltpu.make_async_copy(src_hbm.at[pl.ds(off, K)], srcs[s], isems[s]).wait()
        pltpu.make_async_copy(dst_hbm.at[pl.ds(off, K)], dsts[s], isems[s]).wait()

    def gather_fire(k, s):
        pltpu.async_copy(x_hbm.at[srcs[s]], rows[k], gsems[k])

    def gather_wait(k, s):
        pltpu.make_async_copy(x_hbm.at[srcs[s]], rows[k], gsems[k]).wait()

    def scatter_fire(k, s):
        pltpu.async_copy(rows[k], acc.at[dsts[s]], ssems[k], add=True)

    def scatter_wait(k, s):
        pltpu.make_async_copy(rows[k], acc.at[dsts[s]], ssems[k]).wait()

    r0 = sid * ROWS_PER_TILE
    t0 = NS * ROWS_PER_TILE

    @pl.when(cid == 0)
    def _init_x():
        pltpu.sync_copy(x_hbm.at[pl.ds(r0, ROWS_PER_TILE)],
                        acc.at[pl.ds(r0, ROWS_PER_TILE)])

        @pl.when(sid == NS - 1)
        def _init_tail():
            pltpu.sync_copy(x_hbm.at[pl.ds(t0, ROWS_TAIL)],
                            acc.at[pl.ds(t0, ROWS_TAIL)])

    @pl.when(cid == 1)
    def _init_zero():
        z = jnp.zeros((16,), jnp.float32)
        for i in range(K):
            for l in range(D_FEAT // 16):
                rows[0][i, pl.ds(l * 16, 16)] = z
        for i in range(ROWS_PER_TILE // K):
            pltpu.sync_copy(rows[0],
                            acc.at[pl.ds(r0 + i * K, K)])
        rest = ROWS_PER_TILE - (ROWS_PER_TILE // K) * K
        pltpu.sync_copy(rows[0].at[pl.ds(0, rest)],
                        acc.at[pl.ds(r0 + ROWS_PER_TILE - rest, rest)])

        @pl.when(sid == NS - 1)
        def _zero_tail():
            pltpu.sync_copy(rows[0].at[pl.ds(0, ROWS_TAIL)],
                            acc.at[pl.ds(t0, ROWS_TAIL)])

    plsc.subcore_barrier()

    for s in range(SL):
        idx_fire(s, base + s * K)
    for k in range(R):
        idx_wait(k, base + k * K)
        gather_fire(k, k)

    def body(t, carry):
        c0 = base + (t * SL) * K

        for m in range(R):
            gather_wait(m, m)
            scatter_fire(m, m)
        for m in range(R):
            scatter_wait(m, m)
            idx_wait(m + R, c0 + (m + R) * K)
            gather_fire(m, m + R)

            @pl.when(t < T - 1)
            def _restage_lo():
                idx_fire(m, c0 + (m + SL) * K)

        for m in range(R):
            gather_wait(m, m + R)
            scatter_fire(m, m + R)
        for m in range(R):
            scatter_wait(m, m + R)

            @pl.when(t < T - 1)
            def _next_hi():
                idx_wait(m, c0 + (m + SL) * K)
                gather_fire(m, m)
                idx_fire(m + R, c0 + (m + R + SL) * K)

        return carry

    lax.fori_loop(0, T, body, 0)

    e0 = base + T * SL * K
    for s in range(NEPI):
        idx_fire(s, e0 + s * K)
    for s in range(R):
        idx_wait(s, e0 + s * K)
        gather_fire(s, s)
    idx_wait(R, e0 + R * K)
    for s in range(R):
        gather_wait(s, s)
        scatter_fire(s, s)
    scatter_wait(0, 0)
    gather_fire(0, R)
    gather_wait(0, R)
    scatter_fire(0, R)
    for s in range(1, R):
        scatter_wait(s, s)
    scatter_wait(0, R)

    plsc.subcore_barrier()
    pltpu.sync_copy(acc.at[pl.ds(r0, ROWS_PER_TILE)],
                    p_hbm.at[cid, pl.ds(r0, ROWS_PER_TILE)])

    @pl.when(sid == NS - 1)
    def _drain_tail():
        t0 = NS * ROWS_PER_TILE
        pltpu.sync_copy(acc.at[pl.ds(t0, ROWS_TAIL)],
                        p_hbm.at[cid, pl.ds(t0, ROWS_TAIL)])


def _combine_body(p_ref, o_ref):
    o_ref[...] = p_ref[0] + p_ref[1]


def kernel(x, edge_index):
    src = edge_index[0].astype(jnp.int32)
    dst = edge_index[1].astype(jnp.int32)

    mesh = plsc.VectorSubcoreMesh(core_axis_name="c", subcore_axis_name="s",
                                  num_cores=NC, num_subcores=NS)
    p = pl.kernel(
        _sc_partial_body,
        out_type=jax.ShapeDtypeStruct((NC, N_NODES, D_FEAT), jnp.float32),
        mesh=mesh,
        scratch_types=(
            [pltpu.VMEM((K,), jnp.int32) for _ in range(2 * SL)]
            + [pltpu.VMEM((K, D_FEAT), jnp.float32) for _ in range(R)]
            + [pltpu.VMEM_SHARED((N_NODES, D_FEAT), jnp.float32)]
            + [pltpu.SemaphoreType.DMA for _ in range(SL + 2 * R)]
        ),
    )(x, src, dst)

    BLK = 400
    out = pl.pallas_call(
        _combine_body,
        out_shape=jax.ShapeDtypeStruct((N_NODES, D_FEAT), jnp.float32),
        grid=(N_NODES // BLK,),
        in_specs=[pl.BlockSpec((NC, BLK, D_FEAT), lambda i: (0, i, 0))],
        out_specs=pl.BlockSpec((BLK, D_FEAT), lambda i: (i, 0)),
    )(p)
    return out


# speedup vs baseline: 1.0664x; 1.0022x over previous
import jax
import jax.numpy as jnp
from jax import lax
from jax.experimental import pallas as pl
from jax.experimental.pallas import tpu as pltpu
from jax.experimental.pallas import tpu_sc as plsc

N_NODES = 10000
D_FEAT = 128
N_EDGES = 320000
NC = 2
NS = 16
NW = NC * NS
EPW = N_EDGES // NW
K = 64
R = 5
SL = 2 * R
NFULL = EPW // K
T = NFULL // SL
NEPI = NFULL - T * SL
REM = EPW - NFULL * K
ROWS_PER_TILE = (N_NODES // NS) // 8 * 8
ROWS_TAIL = N_NODES - NS * ROWS_PER_TILE


def _sc_partial_body(x_hbm, src_hbm, dst_hbm, p_hbm, *refs):
    srcs = refs[0:SL]
    dsts = refs[SL:2 * SL]
    rows = refs[2 * SL:2 * SL + R]
    srcr, dstr = refs[2 * SL + R:2 * SL + R + 2]
    acc = refs[2 * SL + R + 2]
    sems = refs[2 * SL + R + 3:]
    isems = sems[0:SL]
    gsems = sems[SL:SL + R]
    ssems = sems[SL + R:SL + 2 * R]

    cid = lax.axis_index("c")
    sid = lax.axis_index("s")
    wid = cid * NS + sid
    base = wid * EPW

    def idx_fire(s, off):
        pltpu.async_copy(src_hbm.at[pl.ds(off, K)], srcs[s], isems[s])
        pltpu.async_copy(dst_hbm.at[pl.ds(off, K)], dsts[s], isems[s])

    def idx_wait(s, off):
        pltpu.make_async_copy(src_hbm.at[pl.ds(off, K)], srcs[s], isems[s]).wait()
        pltpu.make_async_copy(dst_hbm.at[pl.ds(off, K)], dsts[s], isems[s]).wait()

    def gather_fire(k, s):
        pltpu.async_copy(x_hbm.at[srcs[s]], rows[k], gsems[k])

    def gather_wait(k, s):
        pltpu.make_async_copy(x_hbm.at[srcs[s]], rows[k], gsems[k]).wait()

    def scatter_fire(k, s):
        pltpu.async_copy(rows[k], acc.at[dsts[s]], ssems[k], add=True)

    def scatter_wait(k, s):
        pltpu.make_async_copy(rows[k], acc.at[dsts[s]], ssems[k]).wait()

    r0 = sid * ROWS_PER_TILE
    t0 = NS * ROWS_PER_TILE

    @pl.when(cid == 0)
    def _init_x():
        pltpu.sync_copy(x_hbm.at[pl.ds(r0, ROWS_PER_TILE)],
                        acc.at[pl.ds(r0, ROWS_PER_TILE)])

        @pl.when(sid == NS - 1)
        def _init_tail():
            pltpu.sync_copy(x_hbm.at[pl.ds(t0, ROWS_TAIL)],
                            acc.at[pl.ds(t0, ROWS_TAIL)])

    @pl.when(cid == 1)
    def _init_zero():
        z = jnp.zeros((16,), jnp.float32)
        for i in range(K):
            for l in range(D_FEAT // 16):
                rows[0][i, pl.ds(l * 16, 16)] = z
        for i in range(ROWS_PER_TILE // K):
            pltpu.sync_copy(rows[0],
                            acc.at[pl.ds(r0 + i * K, K)])
        rest = ROWS_PER_TILE - (ROWS_PER_TILE // K) * K
        pltpu.sync_copy(rows[0].at[pl.ds(0, rest)],
                        acc.at[pl.ds(r0 + ROWS_PER_TILE - rest, rest)])

        @pl.when(sid == NS - 1)
        def _zero_tail():
            pltpu.sync_copy(rows[0].at[pl.ds(0, ROWS_TAIL)],
                            acc.at[pl.ds(t0, ROWS_TAIL)])

    plsc.subcore_barrier()

    for s in range(SL):
        idx_fire(s, base + s * K)
    for k in range(R):
        idx_wait(k, base + k * K)
        gather_fire(k, k)

    def body(t, carry):
        c0 = base + (t * SL) * K

        for m in range(R):
            gather_wait(m, m)
            scatter_fire(m, m)
        for m in range(R):
            scatter_wait(m, m)
            idx_wait(m + R, c0 + (m + R) * K)
            gather_fire(m, m + R)

            @pl.when(t < T - 1)
            def _restage_lo():
                idx_fire(m, c0 + (m + SL) * K)

        for m in range(R):
            gather_wait(m, m + R)
            scatter_fire(m, m + R)
        for m in range(R):
            scatter_wait(m, m + R)

            @pl.when(t < T - 1)
            def _next_hi():
                idx_wait(m, c0 + (m + SL) * K)
                gather_fire(m, m)
                idx_fire(m + R, c0 + (m + R + SL) * K)

        return carry

    lax.fori_loop(0, T, body, 0)

    e0 = base + T * SL * K
    for s in range(NEPI):
        idx_fire(s, e0 + s * K)
    for s in range(R):
        idx_wait(s, e0 + s * K)
        gather_fire(s, s)
    idx_wait(R, e0 + R * K)
    for s in range(R):
        gather_wait(s, s)
        scatter_fire(s, s)
    scatter_wait(0, 0)
    gather_fire(0, R)
    gather_wait(0, R)
    scatter_fire(0, R)
    for s in range(1, R):
        scatter_wait(s, s)
    scatter_wait(0, R)

    if REM:
        off = base + NFULL * K
        pltpu.sync_copy(src_hbm.at[pl.ds(off, REM)], srcr)
        pltpu.sync_copy(dst_hbm.at[pl.ds(off, REM)], dstr)
        pltpu.async_copy(x_hbm.at[srcr], rows[0].at[pl.ds(0, REM)],
                         gsems[0]).wait()
        pltpu.sync_copy(rows[0].at[pl.ds(0, REM)], acc.at[dstr], add=True)

    plsc.subcore_barrier()
    pltpu.sync_copy(acc.at[pl.ds(r0, ROWS_PER_TILE)],
                    p_hbm.at[cid, pl.ds(r0, ROWS_PER_TILE)])

    @pl.when(sid == NS - 1)
    def _drain_tail():
        t0 = NS * ROWS_PER_TILE
        pltpu.sync_copy(acc.at[pl.ds(t0, ROWS_TAIL)],
                        p_hbm.at[cid, pl.ds(t0, ROWS_TAIL)])


def _combine_body(p_ref, o_ref):
    o_ref[...] = p_ref[0] + p_ref[1]


def kernel(x, edge_index):
    src = edge_index[0].astype(jnp.int32)
    dst = edge_index[1].astype(jnp.int32)

    mesh = plsc.VectorSubcoreMesh(core_axis_name="c", subcore_axis_name="s",
                                  num_cores=NC, num_subcores=NS)
    p = pl.kernel(
        _sc_partial_body,
        out_type=jax.ShapeDtypeStruct((NC, N_NODES, D_FEAT), jnp.float32),
        mesh=mesh,
        scratch_types=(
            [pltpu.VMEM((K,), jnp.int32) for _ in range(2 * SL)]
            + [pltpu.VMEM((K, D_FEAT), jnp.float32) for _ in range(R)]
            + [pltpu.VMEM((REM,), jnp.int32) for _ in range(2)]
            + [pltpu.VMEM_SHARED((N_NODES, D_FEAT), jnp.float32)]
            + [pltpu.SemaphoreType.DMA for _ in range(SL + 2 * R)]
        ),
    )(x, src, dst)

    BLK = 400
    out = pl.pallas_call(
        _combine_body,
        out_shape=jax.ShapeDtypeStruct((N_NODES, D_FEAT), jnp.float32),
        grid=(N_NODES // BLK,),
        in_specs=[pl.BlockSpec((NC, BLK, D_FEAT), lambda i: (0, i, 0))],
        out_specs=pl.BlockSpec((BLK, D_FEAT), lambda i: (i, 0)),
    )(p)
    return out


# speedup vs baseline: 1.1341x; 1.0635x over previous
import jax
import jax.numpy as jnp
from jax import lax
from jax.experimental import pallas as pl
from jax.experimental.pallas import tpu as pltpu
from jax.experimental.pallas import tpu_sc as plsc

N_NODES = 10000
D_FEAT = 128
N_EDGES = 320000
NC = 2
NS = 16
NW = NC * NS
EPW = N_EDGES // NW
K = 64
R = 5
SL = 2 * R
NFULL = EPW // K
T = NFULL // SL
NEPI = NFULL - T * SL
REM = EPW - NFULL * K
ROWS_PER_TILE = (N_NODES // NS) // 8 * 8
ROWS_TAIL = N_NODES - NS * ROWS_PER_TILE


def _sc_partial_body(x_hbm, src_hbm, dst_hbm, p_hbm, *refs):
    srcs = refs[0:SL]
    dsts = refs[SL:2 * SL]
    rows = refs[2 * SL:2 * SL + R]
    srcr, dstr = refs[2 * SL + R:2 * SL + R + 2]
    acc = refs[2 * SL + R + 2]
    sems = refs[2 * SL + R + 3:]
    isems = sems[0:SL]
    gsems = sems[SL:SL + R]
    ssems = sems[SL + R:SL + 2 * R]

    cid = lax.axis_index("c")
    sid = lax.axis_index("s")
    wid = cid * NS + sid
    base = wid * EPW

    def idx_fire(s, off):
        pltpu.async_copy(src_hbm.at[pl.ds(off, K)], srcs[s], isems[s])
        pltpu.async_copy(dst_hbm.at[pl.ds(off, K)], dsts[s], isems[s])

    def idx_wait(s, off):
        pltpu.make_async_copy(src_hbm.at[pl.ds(off, K)], srcs[s], isems[s]).wait()
        pltpu.make_async_copy(dst_hbm.at[pl.ds(off, K)], dsts[s], isems[s]).wait()

    def gather_fire(k, s):
        pltpu.async_copy(x_hbm.at[srcs[s]], rows[k], gsems[k])

    def gather_wait(k, s):
        pltpu.make_async_copy(x_hbm.at[srcs[s]], rows[k], gsems[k]).wait()

    def scatter_fire(k, s):
        pltpu.async_copy(rows[k], acc.at[dsts[s]], ssems[k], add=True)

    def scatter_wait(k, s):
        pltpu.make_async_copy(rows[k], acc.at[dsts[s]], ssems[k]).wait()

    r0 = sid * ROWS_PER_TILE
    t0 = NS * ROWS_PER_TILE

    @pl.when(cid == 0)
    def _init_x():
        pltpu.sync_copy(x_hbm.at[pl.ds(r0, ROWS_PER_TILE)],
                        acc.at[pl.ds(r0, ROWS_PER_TILE)])

        @pl.when(sid == NS - 1)
        def _init_tail():
            pltpu.sync_copy(x_hbm.at[pl.ds(t0, ROWS_TAIL)],
                            acc.at[pl.ds(t0, ROWS_TAIL)])

    @pl.when(cid == 1)
    def _init_zero():
        z = jnp.zeros((16,), jnp.float32)
        for i in range(K):
            for l in range(D_FEAT // 16):
                rows[0][i, pl.ds(l * 16, 16)] = z
        for i in range(ROWS_PER_TILE // K):
            pltpu.sync_copy(rows[0],
                            acc.at[pl.ds(r0 + i * K, K)])
        rest = ROWS_PER_TILE - (ROWS_PER_TILE // K) * K
        pltpu.sync_copy(rows[0].at[pl.ds(0, rest)],
                        acc.at[pl.ds(r0 + ROWS_PER_TILE - rest, rest)])

        @pl.when(sid == NS - 1)
        def _zero_tail():
            pltpu.sync_copy(rows[0].at[pl.ds(0, ROWS_TAIL)],
                            acc.at[pl.ds(t0, ROWS_TAIL)])

    plsc.subcore_barrier()

    for s in range(SL):
        idx_fire(s, base + s * K)
    for k in range(R):
        idx_wait(k, base + k * K)
        gather_fire(k, k)

    def body(t, carry):
        c0 = base + (t * SL) * K

        for m in range(R):
            gather_wait(m, m)
            scatter_fire(m, m)
        for m in range(R):
            scatter_wait(m, m)
            idx_wait(m + R, c0 + (m + R) * K)
            gather_fire(m, m + R)

            @pl.when(t < T - 1)
            def _restage_lo():
                idx_fire(m, c0 + (m + SL) * K)

        for m in range(R):
            gather_wait(m, m + R)
            scatter_fire(m, m + R)
        for m in range(R):
            scatter_wait(m, m + R)

            @pl.when(t < T - 1)
            def _next_hi():
                idx_wait(m, c0 + (m + SL) * K)
                gather_fire(m, m)
                idx_fire(m + R, c0 + (m + R + SL) * K)

        return carry

    lax.fori_loop(0, T, body, 0)

    e0 = base + T * SL * K
    for s in range(NEPI):
        idx_fire(s, e0 + s * K)
    for s in range(R):
        idx_wait(s, e0 + s * K)
        gather_fire(s, s)
    idx_wait(R, e0 + R * K)
    for s in range(R):
        gather_wait(s, s)
        scatter_fire(s, s)
    scatter_wait(0, 0)
    gather_fire(0, R)
    gather_wait(0, R)
    scatter_fire(0, R)
    for s in range(1, R):
        scatter_wait(s, s)
    scatter_wait(0, R)

    if REM:
        off = base + NFULL * K
        pltpu.sync_copy(src_hbm.at[pl.ds(off, REM)], srcr)
        pltpu.sync_copy(dst_hbm.at[pl.ds(off, REM)], dstr)
        pltpu.async_copy(x_hbm.at[srcr], rows[0].at[pl.ds(0, REM)],
                         gsems[0]).wait()
        pltpu.sync_copy(rows[0].at[pl.ds(0, REM)], acc.at[dstr], add=True)

    plsc.subcore_barrier()
    pltpu.sync_copy(acc.at[pl.ds(r0, ROWS_PER_TILE)],
                    p_hbm.at[cid, pl.ds(r0, ROWS_PER_TILE)])

    @pl.when(sid == NS - 1)
    def _drain_tail():
        t0 = NS * ROWS_PER_TILE
        pltpu.sync_copy(acc.at[pl.ds(t0, ROWS_TAIL)],
                        p_hbm.at[cid, pl.ds(t0, ROWS_TAIL)])


def _combine_body(p_ref, o_ref):
    o_ref[...] = p_ref[0] + p_ref[1]


def kernel(x, edge_index):
    src = edge_index[0].astype(jnp.int32)
    dst = edge_index[1].astype(jnp.int32)

    mesh = plsc.VectorSubcoreMesh(core_axis_name="c", subcore_axis_name="s",
                                  num_cores=NC, num_subcores=NS)
    p = pl.kernel(
        _sc_partial_body,
        out_type=jax.ShapeDtypeStruct((NC, N_NODES, D_FEAT), jnp.float32),
        mesh=mesh,
        scratch_types=(
            [pltpu.VMEM((K,), jnp.int32) for _ in range(2 * SL)]
            + [pltpu.VMEM((K, D_FEAT), jnp.float32) for _ in range(R)]
            + [pltpu.VMEM((REM,), jnp.int32) for _ in range(2)]
            + [pltpu.VMEM_SHARED((N_NODES, D_FEAT), jnp.float32)]
            + [pltpu.SemaphoreType.DMA for _ in range(SL + 2 * R)]
        ),
    )(x, src, dst)

    BLK = 2000
    out = pl.pallas_call(
        _combine_body,
        out_shape=jax.ShapeDtypeStruct((N_NODES, D_FEAT), jnp.float32),
        grid=(N_NODES // BLK,),
        in_specs=[pl.BlockSpec((NC, BLK, D_FEAT), lambda i: (0, i, 0))],
        out_specs=pl.BlockSpec((BLK, D_FEAT), lambda i: (i, 0)),
    )(p)
    return out


# speedup vs baseline: 1.1397x; 1.0049x over previous
import jax
import jax.numpy as jnp
from jax import lax
from jax.experimental import pallas as pl
from jax.experimental.pallas import tpu as pltpu
from jax.experimental.pallas import tpu_sc as plsc

N_NODES = 10000
D_FEAT = 128
N_EDGES = 320000
NC = 2
NS = 16
NW = NC * NS
EPW = N_EDGES // NW
K = 64
R = 5
SL = 2 * R
NFULL = EPW // K
T = NFULL // SL
NEPI = NFULL - T * SL
REM = EPW - NFULL * K
ROWS_PER_TILE = (N_NODES // NS) // 8 * 8
ROWS_TAIL = N_NODES - NS * ROWS_PER_TILE


def _sc_partial_body(x_hbm, src_hbm, dst_hbm, p_hbm, *refs):
    srcs = refs[0:SL]
    dsts = refs[SL:2 * SL]
    rows = refs[2 * SL:2 * SL + R]
    srcr, dstr = refs[2 * SL + R:2 * SL + R + 2]
    acc = refs[2 * SL + R + 2]
    sems = refs[2 * SL + R + 3:]
    isems = sems[0:SL]
    gsems = sems[SL:SL + R]
    ssems = sems[SL + R:SL + 2 * R]

    cid = lax.axis_index("c")
    sid = lax.axis_index("s")
    wid = cid * NS + sid
    base = wid * EPW

    def idx_fire(s, off):
        pltpu.async_copy(src_hbm.at[pl.ds(off, K)], srcs[s], isems[s])
        pltpu.async_copy(dst_hbm.at[pl.ds(off, K)], dsts[s], isems[s])

    def idx_wait(s, off):
        pltpu.make_async_copy(src_hbm.at[pl.ds(off, K)], srcs[s], isems[s]).wait()
        pltpu.make_async_copy(dst_hbm.at[pl.ds(off, K)], dsts[s], isems[s]).wait()

    def gather_fire(k, s):
        pltpu.async_copy(x_hbm.at[srcs[s]], rows[k], gsems[k])

    def gather_wait(k, s):
        pltpu.make_async_copy(x_hbm.at[srcs[s]], rows[k], gsems[k]).wait()

    def scatter_fire(k, s):
        pltpu.async_copy(rows[k], acc.at[dsts[s]], ssems[k], add=True)

    def scatter_wait(k, s):
        pltpu.make_async_copy(rows[k], acc.at[dsts[s]], ssems[k]).wait()

    r0 = sid * ROWS_PER_TILE
    t0 = NS * ROWS_PER_TILE

    @pl.when(cid == 0)
    def _init_x():
        pltpu.sync_copy(x_hbm.at[pl.ds(r0, ROWS_PER_TILE)],
                        acc.at[pl.ds(r0, ROWS_PER_TILE)])

        @pl.when(sid == NS - 1)
        def _init_tail():
            pltpu.sync_copy(x_hbm.at[pl.ds(t0, ROWS_TAIL)],
                            acc.at[pl.ds(t0, ROWS_TAIL)])

    @pl.when(cid == 1)
    def _init_zero():
        z = jnp.zeros((16,), jnp.float32)
        for i in range(K):
            for l in range(D_FEAT // 16):
                rows[0][i, pl.ds(l * 16, 16)] = z
        for i in range(ROWS_PER_TILE // K):
            pltpu.sync_copy(rows[0],
                            acc.at[pl.ds(r0 + i * K, K)])
        rest = ROWS_PER_TILE - (ROWS_PER_TILE // K) * K
        pltpu.sync_copy(rows[0].at[pl.ds(0, rest)],
                        acc.at[pl.ds(r0 + ROWS_PER_TILE - rest, rest)])

        @pl.when(sid == NS - 1)
        def _zero_tail():
            pltpu.sync_copy(rows[0].at[pl.ds(0, ROWS_TAIL)],
                            acc.at[pl.ds(t0, ROWS_TAIL)])

    plsc.subcore_barrier()

    for s in range(SL):
        idx_fire(s, base + s * K)
    for k in range(R):
        idx_wait(k, base + k * K)
        gather_fire(k, k)

    def body(t, carry):
        c0 = base + (t * SL) * K

        for m in range(R):
            gather_wait(m, m)
            scatter_fire(m, m)
        for m in range(R):
            scatter_wait(m, m)
            idx_wait(m + R, c0 + (m + R) * K)
            gather_fire(m, m + R)

            @pl.when(t < T - 1)
            def _restage_lo():
                idx_fire(m, c0 + (m + SL) * K)

        for m in range(R):
            gather_wait(m, m + R)
            scatter_fire(m, m + R)
        for m in range(R):
            scatter_wait(m, m + R)

            @pl.when(t < T - 1)
            def _next_hi():
                idx_wait(m, c0 + (m + SL) * K)
                gather_fire(m, m)
                idx_fire(m + R, c0 + (m + R + SL) * K)

        return carry

    lax.fori_loop(0, T, body, 0)

    e0 = base + T * SL * K
    for s in range(NEPI):
        idx_fire(s, e0 + s * K)
    for s in range(R):
        idx_wait(s, e0 + s * K)
        gather_fire(s, s)
    idx_wait(R, e0 + R * K)
    for s in range(R):
        gather_wait(s, s)
        scatter_fire(s, s)
    scatter_wait(0, 0)
    gather_fire(0, R)
    gather_wait(0, R)
    scatter_fire(0, R)
    for s in range(1, R):
        scatter_wait(s, s)
    scatter_wait(0, R)

    if REM:
        off = base + NFULL * K
        pltpu.sync_copy(src_hbm.at[pl.ds(off, REM)], srcr)
        pltpu.sync_copy(dst_hbm.at[pl.ds(off, REM)], dstr)
        pltpu.async_copy(x_hbm.at[srcr], rows[0].at[pl.ds(0, REM)],
                         gsems[0]).wait()
        pltpu.sync_copy(rows[0].at[pl.ds(0, REM)], acc.at[dstr], add=True)

    plsc.subcore_barrier()
    pltpu.sync_copy(acc.at[pl.ds(r0, ROWS_PER_TILE)],
                    p_hbm.at[cid, pl.ds(r0, ROWS_PER_TILE)])

    @pl.when(sid == NS - 1)
    def _drain_tail():
        t0 = NS * ROWS_PER_TILE
        pltpu.sync_copy(acc.at[pl.ds(t0, ROWS_TAIL)],
                        p_hbm.at[cid, pl.ds(t0, ROWS_TAIL)])


def _combine_body(p_ref, o_ref):
    o_ref[...] = p_ref[0] + p_ref[1]


def kernel(x, edge_index):
    src = edge_index[0].astype(jnp.int32)
    dst = edge_index[1].astype(jnp.int32)

    mesh = plsc.VectorSubcoreMesh(core_axis_name="c", subcore_axis_name="s",
                                  num_cores=NC, num_subcores=NS)
    p = pl.kernel(
        _sc_partial_body,
        out_type=jax.ShapeDtypeStruct((NC, N_NODES, D_FEAT), jnp.float32),
        mesh=mesh,
        scratch_types=(
            [pltpu.VMEM((K,), jnp.int32) for _ in range(2 * SL)]
            + [pltpu.VMEM((K, D_FEAT), jnp.float32) for _ in range(R)]
            + [pltpu.VMEM((REM,), jnp.int32) for _ in range(2)]
            + [pltpu.VMEM_SHARED((N_NODES, D_FEAT), jnp.float32)]
            + [pltpu.SemaphoreType.DMA for _ in range(SL + 2 * R)]
        ),
    )(x, src, dst)

    BLK = 5000
    out = pl.pallas_call(
        _combine_body,
        out_shape=jax.ShapeDtypeStruct((N_NODES, D_FEAT), jnp.float32),
        grid=(N_NODES // BLK,),
        in_specs=[pl.BlockSpec((NC, BLK, D_FEAT), lambda i: (0, i, 0))],
        out_specs=pl.BlockSpec((BLK, D_FEAT), lambda i: (i, 0)),
    )(p)
    return out
